# baseline (speedup 1.0000x reference)
import jax
import jax.numpy as jnp
from jax.experimental import pallas as pl

NODE_DIM = 128
EDGE_DIM = 16
HIDDEN = 384
DIM_OUT = 128
BLK = 1280


def _mlp_block(xj_ref, xi_ref, ea_ref, w1a_ref, w1b_ref, w1c_ref, b1_ref,
               w2_ref, b2_ref, out_ref):
    h = jnp.dot(xj_ref[:], w1a_ref[:], preferred_element_type=jnp.float32)
    h = h + jnp.dot(xi_ref[:], w1b_ref[:], preferred_element_type=jnp.float32)
    h = h + jnp.dot(ea_ref[:], w1c_ref[:], preferred_element_type=jnp.float32)
    h = jnp.maximum(h + b1_ref[:], 0.0)
    out_ref[:] = jnp.dot(h, w2_ref[:],
                         preferred_element_type=jnp.float32) + b2_ref[:]


def kernel(x_i, x_j, edge_attr, edge_index, num_nodes, W1, b1, W2, b2):
    del edge_index, num_nodes
    n_edges = x_i.shape[0]
    grid = (n_edges // BLK,)
    w1a = W1[:NODE_DIM]
    w1b = W1[NODE_DIM:2 * NODE_DIM]
    w1c = W1[2 * NODE_DIM:]
    b1r = b1.reshape(1, HIDDEN)
    b2r = b2.reshape(1, DIM_OUT)
    return pl.pallas_call(
        _mlp_block,
        grid=grid,
        in_specs=[
            pl.BlockSpec((BLK, NODE_DIM), lambda i: (i, 0)),
            pl.BlockSpec((BLK, NODE_DIM), lambda i: (i, 0)),
            pl.BlockSpec((BLK, EDGE_DIM), lambda i: (i, 0)),
            pl.BlockSpec((NODE_DIM, HIDDEN), lambda i: (0, 0)),
            pl.BlockSpec((NODE_DIM, HIDDEN), lambda i: (0, 0)),
            pl.BlockSpec((EDGE_DIM, HIDDEN), lambda i: (0, 0)),
            pl.BlockSpec((1, HIDDEN), lambda i: (0, 0)),
            pl.BlockSpec((HIDDEN, DIM_OUT), lambda i: (0, 0)),
            pl.BlockSpec((1, DIM_OUT), lambda i: (0, 0)),
        ],
        out_specs=pl.BlockSpec((BLK, DIM_OUT), lambda i: (i, 0)),
        out_shape=jax.ShapeDtypeStruct((n_edges, DIM_OUT), jnp.float32),
    )(x_j, x_i, edge_attr, w1a, w1b, w1c, b1r, W2, b2r)
